# baseline (speedup 1.0000x reference)
import functools

import jax
import jax.numpy as jnp
from jax import lax
from jax.experimental import pallas as pl
from jax.experimental.pallas import tpu as pltpu
from jax.experimental.pallas import tpu_sc as plsc


def _sc_info():
    info = plsc.get_sparse_core_info()
    return info.num_cores, info.num_subcores


def _make_sc_hr(d: int, n_rows: int):
    nc, ns = _sc_info()
    nw = nc * ns
    b_per_w = n_rows // nw
    mesh = plsc.VectorSubcoreMesh(core_axis_name="c", subcore_axis_name="s")

    @functools.partial(
        pl.kernel,
        mesh=mesh,
        out_type=jax.ShapeDtypeStruct((n_rows, d), jnp.float32),
        scratch_types=[
            pltpu.VMEM((b_per_w,), jnp.int32),
            pltpu.VMEM((b_per_w, d), jnp.float32),
            pltpu.VMEM((b_per_w, d), jnp.float32),
            pltpu.SemaphoreType.DMA,
        ],
    )
    def hr_kernel(table_hbm, idx_hbm, head_hbm, out_hbm, idx_v, rows_v,
                  head_v, sem):
        wid = lax.axis_index("s") * nc + lax.axis_index("c")
        base = wid * b_per_w
        pltpu.sync_copy(idx_hbm.at[pl.ds(base, b_per_w)], idx_v)
        gat = pltpu.async_copy(table_hbm.at[idx_v], rows_v, sem)
        pltpu.sync_copy(head_hbm.at[pl.ds(base, b_per_w)], head_v)
        gat.wait()

        def body(r, carry):
            for j in range(d // 16):
                sl = pl.ds(j * 16, 16)
                rows_v[r, sl] = rows_v[r, sl] * head_v[r, sl]
            return carry

        lax.fori_loop(0, b_per_w, body, 0)
        pltpu.sync_copy(rows_v, out_hbm.at[pl.ds(base, b_per_w)])

    return hr_kernel


def _lane_shuffle(x, idx16):
    dnums = lax.GatherDimensionNumbers(
        offset_dims=(), collapsed_slice_dims=(0,), start_index_map=(0,))
    return lax.gather(x, idx16[:, None], dnums, (1,),
                      mode=lax.GatherScatterMode.PROMISE_IN_BOUNDS)


def _make_sc_score(b_start: int, b_sc: int, n_neg: int, d: int):
    nc, ns = _sc_info()
    nw = nc * ns
    b_per_w = b_sc // nw
    chunk = 4
    n_chunks = b_per_w // chunk
    n_pairs = n_chunks // 2
    mesh = plsc.VectorSubcoreMesh(core_axis_name="c", subcore_axis_name="s")

    @functools.partial(
        pl.kernel,
        mesh=mesh,
        out_type=jax.ShapeDtypeStruct((b_sc, n_neg), jnp.float32),
        scratch_types=[
            pltpu.VMEM((b_per_w,), jnp.int32),
            pltpu.VMEM((b_per_w, d), jnp.float32),
            pltpu.VMEM((b_per_w, d), jnp.float32),
            pltpu.VMEM((chunk, n_neg, d), jnp.float32),
            pltpu.VMEM((chunk, n_neg, d), jnp.float32),
            pltpu.VMEM((b_per_w, n_neg), jnp.float32),
            pltpu.SemaphoreType.DMA,
            pltpu.SemaphoreType.DMA,
            pltpu.SemaphoreType.DMA,
        ],
    )
    def score_kernel(head_hbm, tail_hbm, idx_hbm, table_hbm, out_hbm,
                     idx_v, rel_v, hr_v, tail0, tail1, score_v,
                     sem0, sem1, semg):
        wid = lax.axis_index("s") * nc + lax.axis_index("c")
        wbase = b_start + wid * b_per_w
        obase = wid * b_per_w

        def start(c, buf, sem):
            pltpu.async_copy(tail_hbm.at[pl.ds(wbase + c * chunk, chunk)],
                             buf, sem)

        def wait(c, buf, sem):
            pltpu.make_async_copy(
                tail_hbm.at[pl.ds(wbase + c * chunk, chunk)], buf, sem).wait()

        start(0, tail0, sem0)
        start(1, tail1, sem1)

        pltpu.sync_copy(idx_hbm.at[pl.ds(wbase, b_per_w)], idx_v)
        gat = pltpu.async_copy(table_hbm.at[idx_v], rel_v, semg)
        pltpu.sync_copy(head_hbm.at[pl.ds(wbase, b_per_w)], hr_v)
        gat.wait()

        def hr_body(r, carry):
            for j in range(d // 16):
                sl = pl.ds(j * 16, 16)
                hr_v[r, sl] = hr_v[r, sl] * rel_v[r, sl]
            return carry

        lax.fori_loop(0, b_per_w, hr_body, 0)

        lanes = lax.broadcasted_iota(jnp.int32, (16,), 0)
        perms = [jnp.bitwise_and(lanes + sh, 15) for sh in (8, 4, 2, 1)]

        def compute_chunk(c, tail_ref):
            def b_body(bb, carry):
                brow = c * chunk + bb
                hrs = [hr_v[brow, pl.ds(j * 16, 16)] for j in range(d // 16)]
                for ng in range(n_neg // 16):
                    row16 = jnp.zeros((16,), jnp.float32)
                    for r in range(16):
                        n = ng * 16 + r
                        acc = tail_ref[bb, n, pl.ds(0, 16)] * hrs[0]
                        for j in range(1, d // 16):
                            acc = acc + tail_ref[bb, n, pl.ds(j * 16, 16)] * hrs[j]
                        for p in perms:
                            acc = acc + _lane_shuffle(acc, p)
                        row16 = jnp.where(lanes == r, acc, row16)
                    score_v[brow, pl.ds(ng * 16, 16)] = row16
                return carry

            lax.fori_loop(0, chunk, b_body, 0)

        def pair_body(cp, carry):
            c0 = 2 * cp
            wait(c0, tail0, sem0)
            compute_chunk(c0, tail0)
            start(c0 + 2, tail0, sem0)
            wait(c0 + 1, tail1, sem1)
            compute_chunk(c0 + 1, tail1)
            start(c0 + 3, tail1, sem1)
            return carry

        lax.fori_loop(0, n_pairs - 1, pair_body, 0)
        c0 = n_chunks - 2
        wait(c0, tail0, sem0)
        compute_chunk(c0, tail0)
        wait(c0 + 1, tail1, sem1)
        compute_chunk(c0 + 1, tail1)

        pltpu.sync_copy(score_v, out_hbm.at[pl.ds(obase, b_per_w)])

    return score_kernel


def _score_body(hr_ref, tail_ref, out_ref):
    hr = hr_ref[...]
    out_ref[...] = jnp.sum(tail_ref[...] * hr[:, None, :], axis=2)


def kernel(head_embs, tail_embs, rel_idx, w_relation):
    b, n_neg, d = tail_embs.shape
    b_sc = 1024
    b_tc = b - b_sc
    idx = rel_idx.astype(jnp.int32)

    score_sc = _make_sc_score(b_tc, b_sc, n_neg, d)(
        head_embs, tail_embs, idx, w_relation)

    hr_tc = _make_sc_hr(d, b_tc)(w_relation, idx, head_embs)

    blk = 512
    score_tc = pl.pallas_call(
        _score_body,
        grid=(b_tc // blk,),
        in_specs=[
            pl.BlockSpec((blk, d), lambda i: (i, 0)),
            pl.BlockSpec((blk, n_neg, d), lambda i: (i, 0, 0)),
        ],
        out_specs=pl.BlockSpec((blk, n_neg), lambda i: (i, 0)),
        out_shape=jax.ShapeDtypeStruct((b_tc, n_neg), jnp.float32),
    )(hr_tc, tail_embs)

    return jnp.concatenate([score_tc, score_sc], axis=0)


# speedup vs baseline: 1.0330x; 1.0330x over previous
import functools

import jax
import jax.numpy as jnp
from jax import lax
from jax.experimental import pallas as pl
from jax.experimental.pallas import tpu as pltpu
from jax.experimental.pallas import tpu_sc as plsc


def _sc_info():
    info = plsc.get_sparse_core_info()
    return info.num_cores, info.num_subcores


def _make_sc_hr(d: int, n_rows: int):
    nc, ns = _sc_info()
    nw = nc * ns
    b_per_w = n_rows // nw
    mesh = plsc.VectorSubcoreMesh(core_axis_name="c", subcore_axis_name="s")

    @functools.partial(
        pl.kernel,
        mesh=mesh,
        out_type=jax.ShapeDtypeStruct((n_rows, d), jnp.float32),
        scratch_types=[
            pltpu.VMEM((b_per_w,), jnp.int32),
            pltpu.VMEM((b_per_w, d), jnp.float32),
            pltpu.VMEM((b_per_w, d), jnp.float32),
            pltpu.SemaphoreType.DMA,
        ],
    )
    def hr_kernel(table_hbm, idx_hbm, head_hbm, out_hbm, idx_v, rows_v,
                  head_v, sem):
        wid = lax.axis_index("s") * nc + lax.axis_index("c")
        base = wid * b_per_w
        pltpu.sync_copy(idx_hbm.at[pl.ds(base, b_per_w)], idx_v)
        gat = pltpu.async_copy(table_hbm.at[idx_v], rows_v, sem)
        pltpu.sync_copy(head_hbm.at[pl.ds(base, b_per_w)], head_v)
        gat.wait()

        def body(r, carry):
            for j in range(d // 16):
                sl = pl.ds(j * 16, 16)
                rows_v[r, sl] = rows_v[r, sl] * head_v[r, sl]
            return carry

        lax.fori_loop(0, b_per_w, body, 0)
        pltpu.sync_copy(rows_v, out_hbm.at[pl.ds(base, b_per_w)])

    return hr_kernel


def _lane_shuffle(x, idx16):
    dnums = lax.GatherDimensionNumbers(
        offset_dims=(), collapsed_slice_dims=(0,), start_index_map=(0,))
    return lax.gather(x, idx16[:, None], dnums, (1,),
                      mode=lax.GatherScatterMode.PROMISE_IN_BOUNDS)


def _make_sc_score(b_start: int, b_sc: int, n_neg: int, d: int):
    nc, ns = _sc_info()
    nw = nc * ns
    b_per_w = b_sc // nw
    chunk = 4
    n_chunks = b_per_w // chunk
    n_pairs = n_chunks // 2
    mesh = plsc.VectorSubcoreMesh(core_axis_name="c", subcore_axis_name="s")

    @functools.partial(
        pl.kernel,
        mesh=mesh,
        out_type=jax.ShapeDtypeStruct((b_sc, n_neg), jnp.float32),
        scratch_types=[
            pltpu.VMEM((b_per_w,), jnp.int32),
            pltpu.VMEM((b_per_w, d), jnp.float32),
            pltpu.VMEM((b_per_w, d), jnp.float32),
            pltpu.VMEM((chunk, n_neg, d), jnp.float32),
            pltpu.VMEM((chunk, n_neg, d), jnp.float32),
            pltpu.VMEM((b_per_w, n_neg), jnp.float32),
            pltpu.SemaphoreType.DMA,
            pltpu.SemaphoreType.DMA,
            pltpu.SemaphoreType.DMA,
        ],
    )
    def score_kernel(head_hbm, tail_hbm, idx_hbm, table_hbm, out_hbm,
                     idx_v, rel_v, hr_v, tail0, tail1, score_v,
                     sem0, sem1, semg):
        wid = lax.axis_index("s") * nc + lax.axis_index("c")
        wbase = b_start + wid * b_per_w
        obase = wid * b_per_w

        def start(c, buf, sem):
            pltpu.async_copy(tail_hbm.at[pl.ds(wbase + c * chunk, chunk)],
                             buf, sem)

        def wait(c, buf, sem):
            pltpu.make_async_copy(
                tail_hbm.at[pl.ds(wbase + c * chunk, chunk)], buf, sem).wait()

        start(0, tail0, sem0)
        start(1, tail1, sem1)

        pltpu.sync_copy(idx_hbm.at[pl.ds(wbase, b_per_w)], idx_v)
        gat = pltpu.async_copy(table_hbm.at[idx_v], rel_v, semg)
        pltpu.sync_copy(head_hbm.at[pl.ds(wbase, b_per_w)], hr_v)
        gat.wait()

        def hr_body(r, carry):
            for j in range(d // 16):
                sl = pl.ds(j * 16, 16)
                hr_v[r, sl] = hr_v[r, sl] * rel_v[r, sl]
            return carry

        lax.fori_loop(0, b_per_w, hr_body, 0)

        lanes = lax.broadcasted_iota(jnp.int32, (16,), 0)
        perms = [jnp.bitwise_and(lanes + sh, 15) for sh in (8, 4, 2, 1)]

        def compute_chunk(c, tail_ref):
            def b_body(bb, carry):
                brow = c * chunk + bb
                hrs = [hr_v[brow, pl.ds(j * 16, 16)] for j in range(d // 16)]
                for ng in range(n_neg // 16):
                    row16 = jnp.zeros((16,), jnp.float32)
                    for r in range(16):
                        n = ng * 16 + r
                        acc = tail_ref[bb, n, pl.ds(0, 16)] * hrs[0]
                        for j in range(1, d // 16):
                            acc = acc + tail_ref[bb, n, pl.ds(j * 16, 16)] * hrs[j]
                        for p in perms:
                            acc = acc + _lane_shuffle(acc, p)
                        row16 = jnp.where(lanes == r, acc, row16)
                    score_v[brow, pl.ds(ng * 16, 16)] = row16
                return carry

            lax.fori_loop(0, chunk, b_body, 0)

        def pair_body(cp, carry):
            c0 = 2 * cp
            wait(c0, tail0, sem0)
            compute_chunk(c0, tail0)
            start(c0 + 2, tail0, sem0)
            wait(c0 + 1, tail1, sem1)
            compute_chunk(c0 + 1, tail1)
            start(c0 + 3, tail1, sem1)
            return carry

        lax.fori_loop(0, n_pairs - 1, pair_body, 0)
        c0 = n_chunks - 2
        wait(c0, tail0, sem0)
        compute_chunk(c0, tail0)
        wait(c0 + 1, tail1, sem1)
        compute_chunk(c0 + 1, tail1)

        pltpu.sync_copy(score_v, out_hbm.at[pl.ds(obase, b_per_w)])

    return score_kernel


def _make_tc_score(b_tc: int, n_neg: int, d: int):
    rows = 256
    nbuf = 3
    n_chunks = b_tc // rows

    def body(hr_hbm, tail_hbm, out_hbm, hr_v, t0, t1, t2, s0, s1,
             hsem, ts0, ts1, ts2, os0, os1):
        tails, tsems = [t0, t1, t2], [ts0, ts1, ts2]
        scores, osems = [s0, s1], [os0, os1]

        def tail_copy(c):
            return pltpu.make_async_copy(
                tail_hbm.at[pl.ds(c * rows, rows)], tails[c % nbuf],
                tsems[c % nbuf])

        def out_copy(c):
            return pltpu.make_async_copy(
                scores[c % 2], out_hbm.at[pl.ds(c * rows, rows)],
                osems[c % 2])

        hcopy = pltpu.make_async_copy(hr_hbm, hr_v, hsem)
        hcopy.start()
        for c in range(min(nbuf, n_chunks)):
            tail_copy(c).start()
        hcopy.wait()
        for c in range(n_chunks):
            tail_copy(c).wait()
            if c >= 2:
                out_copy(c - 2).wait()
            hr_blk = hr_v[pl.ds(c * rows, rows), :]
            scores[c % 2][...] = jnp.sum(
                tails[c % nbuf][...] * hr_blk[:, None, :], axis=2)
            out_copy(c).start()
            if c + nbuf < n_chunks:
                tail_copy(c + nbuf).start()
        out_copy(n_chunks - 2).wait()
        out_copy(n_chunks - 1).wait()

    return pl.pallas_call(
        body,
        in_specs=[
            pl.BlockSpec(memory_space=pl.ANY),
            pl.BlockSpec(memory_space=pl.ANY),
        ],
        out_specs=pl.BlockSpec(memory_space=pl.ANY),
        out_shape=jax.ShapeDtypeStruct((b_tc, n_neg), jnp.float32),
        scratch_shapes=[
            pltpu.VMEM((b_tc, d), jnp.float32),
            pltpu.VMEM((rows, n_neg, d), jnp.float32),
            pltpu.VMEM((rows, n_neg, d), jnp.float32),
            pltpu.VMEM((rows, n_neg, d), jnp.float32),
            pltpu.VMEM((rows, n_neg), jnp.float32),
            pltpu.VMEM((rows, n_neg), jnp.float32),
            pltpu.SemaphoreType.DMA,
            pltpu.SemaphoreType.DMA,
            pltpu.SemaphoreType.DMA,
            pltpu.SemaphoreType.DMA,
            pltpu.SemaphoreType.DMA,
            pltpu.SemaphoreType.DMA,
        ],
    )


def kernel(head_embs, tail_embs, rel_idx, w_relation):
    b, n_neg, d = tail_embs.shape
    b_sc = 1024
    b_tc = b - b_sc
    idx = rel_idx.astype(jnp.int32)

    score_sc = _make_sc_score(b_tc, b_sc, n_neg, d)(
        head_embs, tail_embs, idx, w_relation)

    hr_tc = _make_sc_hr(d, b_tc)(w_relation, idx, head_embs)

    score_tc = _make_tc_score(b_tc, n_neg, d)(hr_tc, tail_embs)

    return jnp.concatenate([score_tc, score_sc], axis=0)
